# baseline (speedup 1.0000x reference)
import functools

import jax
import jax.numpy as jnp
from jax import lax
from jax.experimental import pallas as pl
from jax.experimental.pallas import tpu as pltpu
from jax.experimental.pallas import tpu_sc as plsc

_N = 10000
_E = 320000
_HID = 256
_HALF = 128
_NG = 64

_NC = 2
_NS = 16
_NW = _NC * _NS

_CH = 128
_CPT = 160
_BLK = 32
_ECH = _CPT * _NS
_EPAD = _ECH * _CH
_PAD_NODE = _N

_NP = 10240
_ROWS_PT = _NP // _NS

_CPW = _ECH // _NW

_BR = 2048
_NB = _NP // _BR
_BRF = 2000
_NBF = _N // _BRF


def _sc_mesh():
    return plsc.VectorSubcoreMesh(
        core_axis_name="c", subcore_axis_name="s",
        num_cores=_NC, num_subcores=_NS,
    )


def _deg_body(dstm_hbm, ones_hbm, zrows_hbm, out0_hbm, out1_hbm,
              didx, onesbuf, acc):
    c = lax.axis_index("c")
    s = lax.axis_index("s")
    wid = c * _NS + s
    r0 = s * _ROWS_PT

    pltpu.sync_copy(ones_hbm, onesbuf)
    pltpu.sync_copy(dstm_hbm.at[pl.ds(wid * _CPW, _CPW)], didx)
    pltpu.sync_copy(zrows_hbm, acc.at[pl.ds(r0, _ROWS_PT)])
    plsc.subcore_barrier()

    def _chunk(k, carry):
        pltpu.sync_copy(onesbuf, acc.at[didx.at[k]], add=True)
        return carry

    lax.fori_loop(0, _CPW, _chunk, 0)
    plsc.subcore_barrier()

    def _writeout(out_hbm):
        pltpu.sync_copy(acc.at[pl.ds(r0, _ROWS_PT)], out_hbm.at[pl.ds(r0, _ROWS_PT)])

    @pl.when(c == 0)
    def _():
        _writeout(out0_hbm)

    @pl.when(c == 1)
    def _():
        _writeout(out1_hbm)


@functools.cache
def _deg_kernel_fn():
    return pl.kernel(
        _deg_body,
        out_type=(
            jax.ShapeDtypeStruct((_NP, _HALF), jnp.float32),
            jax.ShapeDtypeStruct((_NP, _HALF), jnp.float32),
        ),
        mesh=_sc_mesh(),
        scratch_types=[
            pltpu.VMEM((_CPW, _CH), jnp.int32),
            pltpu.VMEM((_CH, _HALF), jnp.float32),
            pltpu.VMEM_SHARED((_NP, _HALF), jnp.float32),
        ],
    )


def _deg_kernel(dstm, ones_rows, zrows):
    return _deg_kernel_fn()(dstm, ones_rows, zrows)


def _agg_body(g0_hbm, g1_hbm, srcm_hbm, dstm_hbm, zrows_hbm,
              out0_hbm, out1_hbm, sidx, didx, buf0, buf1, acc, gsem):
    c = lax.axis_index("c")
    s = lax.axis_index("s")
    r0 = s * _ROWS_PT
    k0 = s * _CPT

    def _run(g_hbm, out_hbm):
        pltpu.sync_copy(zrows_hbm, acc.at[pl.ds(r0, _ROWS_PT)])
        plsc.subcore_barrier()

        bufs = (buf0, buf1)
        for stg in range(_CPT // _BLK):
            base = k0 + stg * _BLK
            pltpu.sync_copy(srcm_hbm.at[pl.ds(base, _BLK)], sidx)
            pltpu.sync_copy(dstm_hbm.at[pl.ds(base, _BLK)], didx)
            pltpu.async_copy(g_hbm.at[sidx.at[0]], buf0, gsem)

            def _round(r, carry):
                for b in range(2):
                    k = r * 2 + b
                    pltpu.make_async_copy(g_hbm.at[sidx.at[0]], bufs[b], gsem).wait()
                    if b == 0:
                        pltpu.async_copy(g_hbm.at[sidx.at[k + 1]], bufs[1], gsem)
                    else:
                        @pl.when(k + 1 < _BLK)
                        def _():
                            pltpu.async_copy(g_hbm.at[sidx.at[k + 1]], bufs[0], gsem)
                    pltpu.sync_copy(bufs[b], acc.at[didx.at[k]], add=True)
                return carry

            lax.fori_loop(0, _BLK // 2, _round, 0)
        plsc.subcore_barrier()
        pltpu.sync_copy(acc.at[pl.ds(r0, _ROWS_PT)], out_hbm.at[pl.ds(r0, _ROWS_PT)])

    @pl.when(c == 0)
    def _():
        _run(g0_hbm, out0_hbm)

    @pl.when(c == 1)
    def _():
        _run(g1_hbm, out1_hbm)


@functools.cache
def _agg_kernel_fn():
    return pl.kernel(
        _agg_body,
        out_type=(
            jax.ShapeDtypeStruct((_NP, _HALF), jnp.float32),
            jax.ShapeDtypeStruct((_NP, _HALF), jnp.float32),
        ),
        mesh=_sc_mesh(),
        scratch_types=[
            pltpu.VMEM((_BLK, _CH), jnp.int32),
            pltpu.VMEM((_BLK, _CH), jnp.int32),
            pltpu.VMEM((_CH, _HALF), jnp.float32),
            pltpu.VMEM((_CH, _HALF), jnp.float32),
            pltpu.VMEM_SHARED((_NP, _HALF), jnp.float32),
            pltpu.SemaphoreType.DMA,
        ],
    )


def _agg_kernel(g0, g1, srcm, dstm, zrows):
    return _agg_kernel_fn()(g0, g1, srcm, dstm, zrows)


def _tc1_body(x_ref, d0_ref, d1_ref, w_ref, g0_ref, g1_ref, dis_ref):
    dis = lax.rsqrt(d0_ref[:, :1] + d1_ref[:, :1] + 1.0)
    h = jnp.dot(x_ref[...], w_ref[...], preferred_element_type=jnp.float32)
    g = h * dis
    g0_ref[...] = g[:, :_HALF]
    g1_ref[...] = g[:, _HALF:]
    dis_ref[...] = dis


def _tc_layer1(x, deg0, deg1, W1):
    return pl.pallas_call(
        _tc1_body,
        grid=(_NB,),
        in_specs=[
            pl.BlockSpec((_BR, _HALF), lambda i: (i, 0)),
            pl.BlockSpec((_BR, _HALF), lambda i: (i, 0)),
            pl.BlockSpec((_BR, _HALF), lambda i: (i, 0)),
            pl.BlockSpec((_HALF, _HID), lambda i: (0, 0)),
        ],
        out_specs=(
            pl.BlockSpec((_BR, _HALF), lambda i: (i, 0)),
            pl.BlockSpec((_BR, _HALF), lambda i: (i, 0)),
            pl.BlockSpec((_BR, 1), lambda i: (i, 0)),
        ),
        out_shape=(
            jax.ShapeDtypeStruct((_NP, _HALF), jnp.float32),
            jax.ShapeDtypeStruct((_NP, _HALF), jnp.float32),
            jax.ShapeDtypeStruct((_NP, 1), jnp.float32),
        ),
        compiler_params=pltpu.CompilerParams(
            dimension_semantics=("parallel",),
        ),
    )(x, deg0, deg1, W1)


def _tcmid_body(a0_ref, a1_ref, g0_ref, g1_ref, dis_ref, b_ref, w_ref,
                ng0_ref, ng1_ref):
    dis = dis_ref[...]
    b = b_ref[...]
    a0 = jnp.maximum(dis * (a0_ref[...] + g0_ref[...]) + b[:, :_HALF], 0.0)
    a1 = jnp.maximum(dis * (a1_ref[...] + g1_ref[...]) + b[:, _HALF:], 0.0)
    h = (jnp.dot(a0, w_ref[:_HALF, :], preferred_element_type=jnp.float32)
         + jnp.dot(a1, w_ref[_HALF:, :], preferred_element_type=jnp.float32))
    g = h * dis
    ng0_ref[...] = g[:, :_HALF]
    ng1_ref[...] = g[:, _HALF:]


def _tc_layer_mid(agg0, agg1, g0, g1, dis, b_prev, W):
    return pl.pallas_call(
        _tcmid_body,
        grid=(_NB,),
        in_specs=[
            pl.BlockSpec((_BR, _HALF), lambda i: (i, 0)),
            pl.BlockSpec((_BR, _HALF), lambda i: (i, 0)),
            pl.BlockSpec((_BR, _HALF), lambda i: (i, 0)),
            pl.BlockSpec((_BR, _HALF), lambda i: (i, 0)),
            pl.BlockSpec((_BR, 1), lambda i: (i, 0)),
            pl.BlockSpec((1, _HID), lambda i: (0, 0)),
            pl.BlockSpec((_HID, _HID), lambda i: (0, 0)),
        ],
        out_specs=(
            pl.BlockSpec((_BR, _HALF), lambda i: (i, 0)),
            pl.BlockSpec((_BR, _HALF), lambda i: (i, 0)),
        ),
        out_shape=(
            jax.ShapeDtypeStruct((_NP, _HALF), jnp.float32),
            jax.ShapeDtypeStruct((_NP, _HALF), jnp.float32),
        ),
        compiler_params=pltpu.CompilerParams(
            dimension_semantics=("parallel",),
        ),
    )(agg0, agg1, g0, g1, dis, b_prev, W)


def _tcfin_body(a0_ref, a1_ref, g0_ref, g1_ref, dis_ref, b_ref, batch_ref,
                wl_ref, bl_ref, out_ref):
    i = pl.program_id(0)
    dis = dis_ref[...]
    b = b_ref[...]
    a0 = jnp.maximum(dis * (a0_ref[...] + g0_ref[...]) + b[:, :_HALF], 0.0)
    a1 = jnp.maximum(dis * (a1_ref[...] + g1_ref[...]) + b[:, _HALF:], 0.0)
    sval = (jnp.dot(a0, wl_ref[:_HALF, :], preferred_element_type=jnp.float32)
            + jnp.dot(a1, wl_ref[_HALF:, :], preferred_element_type=jnp.float32))
    bt = batch_ref[0]
    m = (lax.broadcasted_iota(jnp.int32, (_NG, _BRF), 0) == bt).astype(jnp.float32)
    contrib = jnp.dot(m, sval, preferred_element_type=jnp.float32)

    @pl.when(i == 0)
    def _():
        out_ref[...] = jnp.broadcast_to(bl_ref[...], (_NG, 1))

    out_ref[...] += contrib


def _tc_final(agg0, agg1, g0, g1, dis, b_prev, batch3d, Wl, bl2d):
    return pl.pallas_call(
        _tcfin_body,
        grid=(_NBF,),
        in_specs=[
            pl.BlockSpec((_BRF, _HALF), lambda i: (i, 0)),
            pl.BlockSpec((_BRF, _HALF), lambda i: (i, 0)),
            pl.BlockSpec((_BRF, _HALF), lambda i: (i, 0)),
            pl.BlockSpec((_BRF, _HALF), lambda i: (i, 0)),
            pl.BlockSpec((_BRF, 1), lambda i: (i, 0)),
            pl.BlockSpec((1, _HID), lambda i: (0, 0)),
            pl.BlockSpec((1, 1, _BRF), lambda i: (i, 0, 0)),
            pl.BlockSpec((_HID, 1), lambda i: (0, 0)),
            pl.BlockSpec((1, 1), lambda i: (0, 0)),
        ],
        out_specs=pl.BlockSpec((_NG, 1), lambda i: (0, 0)),
        out_shape=jax.ShapeDtypeStruct((_NG, 1), jnp.float32),
        compiler_params=pltpu.CompilerParams(
            dimension_semantics=("arbitrary",),
        ),
    )(agg0, agg1, g0, g1, dis, b_prev, batch3d, Wl, bl2d)


def kernel(x, edge_index, batch, W1, b1, W2, b2, W3, b3, W4, b4, Wl, bl):
    pad = jnp.full((_EPAD - _E,), _PAD_NODE, jnp.int32)
    srcm = jnp.concatenate([edge_index[0].astype(jnp.int32), pad]).reshape(_ECH, _CH)
    dstm = jnp.concatenate([edge_index[1].astype(jnp.int32), pad]).reshape(_ECH, _CH)
    batch3d = batch.astype(jnp.int32).reshape(_NBF, 1, _BRF)
    xp = jnp.pad(x, ((0, _NP - _N), (0, 0)))

    zrows = jnp.zeros((_ROWS_PT, _HALF), jnp.float32)
    ones_rows = jnp.ones((_CH, _HALF), jnp.float32)

    deg0, deg1 = _deg_kernel(dstm, ones_rows, zrows)
    g0, g1, dis = _tc_layer1(xp, deg0, deg1, W1)

    for b_prev, W in ((b1, W2), (b2, W3), (b3, W4)):
        agg0, agg1 = _agg_kernel(g0, g1, srcm, dstm, zrows)
        g0, g1 = _tc_layer_mid(agg0, agg1, g0, g1, dis,
                               b_prev.reshape(1, _HID), W)

    agg0, agg1 = _agg_kernel(g0, g1, srcm, dstm, zrows)
    out = _tc_final(agg0, agg1, g0, g1, dis, b4.reshape(1, _HID), batch3d,
                    Wl, bl.reshape(1, 1))
    return out
